# baseline (speedup 1.0000x reference)
import functools

import jax
import jax.numpy as jnp
from jax import lax
from jax.experimental import pallas as pl
from jax.experimental.pallas import tpu as pltpu

_NEGPOS_RATIO = 3
_PBLK = 2048


def _p1_body(conf_ref, lab_ref, labr_ref, loc_ref, gt_ref, lc_ref, st_ref,
             *, P, C, pblk):
    p = pl.program_id(1)
    f32 = jnp.float32
    conf = conf_ref[0]
    labL = lab_ref[0]
    lab2 = labr_ref[0][:, 0:1]

    dn = (((1,), (1,)), ((), ()))
    ones_c = jnp.ones((1, C), f32)
    sumexp = lax.dot_general(ones_c, jnp.exp(conf), dn, preferred_element_type=f32)
    iota_c = lax.broadcasted_iota(jnp.int32, (pblk, C), 1)
    onehot = iota_c == lab2
    gathered = lax.dot_general(ones_c, jnp.where(onehot, conf, 0.0), dn,
                               preferred_element_type=f32)
    zlab = lax.dot_general(jnp.ones((1, 1), f32), (lab2 == 0).astype(f32), dn,
                           preferred_element_type=f32)

    lane_i = lax.broadcasted_iota(jnp.int32, (1, pblk), 1)
    valid_l = (p * pblk + lane_i) < P
    pos_l = zlab < 0.5
    ce = jnp.log(sumexp) - gathered
    lc = jnp.where(valid_l & jnp.logical_not(pos_l), jnp.maximum(ce, 0.0), 0.0)
    lc_ref[...] = jnp.reshape(lc, (1, 1, 1, pblk))

    posmask = valid_l & pos_l
    np_blk = jnp.sum(jnp.where(posmask, 1.0, 0.0))
    nz_blk = jnp.sum(jnp.where(lc > 0.0, 1.0, 0.0))
    tot_blk = jnp.sum(lc)
    posce_blk = jnp.sum(jnp.where(posmask, ce, 0.0))

    nrow = pblk // 32
    posLf = (labL > 0.5).astype(f32)
    iota_j = lax.broadcasted_iota(jnp.int32, (128, 128), 0)
    iota_l4 = lax.broadcasted_iota(jnp.int32, (128, 128), 1) // 4
    parts = []
    for g in range(4):
        Rg = (iota_j == (32 * g + iota_l4)).astype(f32)
        parts.append(lax.dot_general(posLf, Rg, (((1,), (0,)), ((), ())),
                                     preferred_element_type=f32))
    posrep = jnp.reshape(jnp.stack(parts, axis=1), (nrow, 128))
    d = loc_ref[0] - gt_ref[0]
    ad = jnp.abs(d)
    sl1 = jnp.where(ad < 1.0, 0.5 * d * d, ad - 0.5)
    sl1_blk = jnp.sum(jnp.where(posrep > 0.5, sl1, 0.0))

    li = lax.broadcasted_iota(jnp.int32, (1, 128), 1)
    vals = jnp.reshape(
        jnp.where(li == 0, np_blk, 0.0)
        + jnp.where(li == 1, nz_blk, 0.0)
        + jnp.where(li == 2, tot_blk, 0.0)
        + jnp.where(li == 3, posce_blk, 0.0)
        + jnp.where(li == 4, sl1_blk, 0.0), (1, 1, 128))

    @pl.when(p == 0)
    def _init():
        st_ref[...] = vals

    @pl.when(p != 0)
    def _acc():
        st_ref[...] += vals


def _p2_body(lc_ref, st_ref, out_ref, *, B, P, Ppad):
    f32, i32 = jnp.float32, jnp.int32
    CH = Ppad if Ppad < 2048 else 2048
    NCH = Ppad // CH
    st = st_ref[...]
    npv = st[:, 0:1]
    Kf = jnp.minimum(float(_NEGPOS_RATIO) * npv, float(P - 1))
    Ki = Kf.astype(i32)

    def cnt_ge(mid):
        def body(c, acc):
            v = lc_ref[:, pl.ds(c * CH, CH)]
            ui = lax.bitcast_convert_type(v, i32)
            return acc + jnp.sum((ui >= mid).astype(i32), axis=1, keepdims=True)
        return lax.fori_loop(0, NCH, body, jnp.zeros((B, 1), i32))

    def bis(_, lohi):
        lo, hi = lohi
        mid = lo + (hi - lo) // 2
        ge = cnt_ge(mid) >= Ki
        return (jnp.where(ge, mid, lo), jnp.where(ge, hi, mid))
    lo, _ = lax.fori_loop(
        0, 31, bis,
        (jnp.zeros((B, 1), i32), jnp.full((B, 1), 2**31 - 1, i32)))
    t = lax.bitcast_convert_type(lo, f32)

    def body2(c, acc):
        s, cg = acc
        v = lc_ref[:, pl.ds(c * CH, CH)]
        gtm = v > t
        s = s + jnp.sum(jnp.where(gtm, v, 0.0), axis=1, keepdims=True)
        cg = cg + jnp.sum(gtm.astype(f32), axis=1, keepdims=True)
        return (s, cg)
    sgt, cgt = lax.fori_loop(0, NCH, body2,
                             (jnp.zeros((B, 1), f32), jnp.zeros((B, 1), f32)))
    S = jnp.where(Ki > 0, sgt + (Kf - cgt) * t, 0.0)

    N = jnp.sum(npv)
    loss = (jnp.sum(st[:, 3:4]) + jnp.sum(st[:, 4:5]) + jnp.sum(S)) / N
    out_ref[...] = jnp.reshape(loss, (1, 1))


def kernel(loc, conf, priors, loc_gt, labels):
    del priors
    B, P, C = conf.shape
    pblk = _PBLK
    NP = (P + pblk - 1) // pblk
    Ppad = NP * pblk

    labf = jnp.reshape(
        jnp.pad(labels[..., 0].astype(jnp.float32), ((0, 0), (0, Ppad - P))),
        (B, Ppad // 128, 128))
    labr = jnp.pad(jnp.broadcast_to(labels.astype(jnp.int32), (B, P, 8)),
                   ((0, 0), (0, Ppad - P), (0, 0)))
    locf = jnp.reshape(
        jnp.pad(jnp.reshape(loc, (B, P * 4)), ((0, 0), (0, (Ppad - P) * 4))),
        (B, Ppad // 32, 128))
    gtf = jnp.reshape(
        jnp.pad(jnp.reshape(loc_gt, (B, P * 4)), ((0, 0), (0, (Ppad - P) * 4))),
        (B, Ppad // 32, 128))

    lc, st = pl.pallas_call(
        functools.partial(_p1_body, P=P, C=C, pblk=pblk),
        grid=(B, NP),
        in_specs=[
            pl.BlockSpec((1, pblk, C), lambda b, p: (b, p, 0)),
            pl.BlockSpec((1, pblk // 128, 128), lambda b, p: (b, p, 0)),
            pl.BlockSpec((1, pblk, 8), lambda b, p: (b, p, 0)),
            pl.BlockSpec((1, pblk // 32, 128), lambda b, p: (b, p, 0)),
            pl.BlockSpec((1, pblk // 32, 128), lambda b, p: (b, p, 0)),
        ],
        out_specs=[
            pl.BlockSpec((1, 1, 1, pblk), lambda b, p: (b, p, 0, 0)),
            pl.BlockSpec((1, 1, 128), lambda b, p: (b, 0, 0)),
        ],
        out_shape=[
            jax.ShapeDtypeStruct((B, NP, 1, pblk), jnp.float32),
            jax.ShapeDtypeStruct((B, 1, 128), jnp.float32),
        ],
        compiler_params=pltpu.CompilerParams(
            dimension_semantics=("arbitrary", "arbitrary")),
    )(conf, labf, labr, locf, gtf)
    lc = jnp.reshape(lc, (B, Ppad))
    st = jnp.reshape(st, (B, 128))

    out = pl.pallas_call(
        functools.partial(_p2_body, B=B, P=P, Ppad=Ppad),
        out_shape=jax.ShapeDtypeStruct((1, 1), jnp.float32),
    )(lc, st)
    return jnp.reshape(out, ())


# speedup vs baseline: 1.5936x; 1.5936x over previous
import functools

import jax
import jax.numpy as jnp
from jax import lax
from jax.experimental import pallas as pl
from jax.experimental.pallas import tpu as pltpu

_NEGPOS_RATIO = 3
_PBLK = 4096


def _p1_body(conf_ref, lab_ref, loc_ref, gt_ref, lc_ref, st_ref, *, P, C, pblk):
    p = pl.program_id(1)
    f32 = jnp.float32
    conf = conf_ref[0]
    labL = lab_ref[0]

    dn = (((1,), (1,)), ((), ()))
    iota_i = lax.broadcasted_iota(jnp.int32, (pblk, pblk // 128), 0)
    iota_s = lax.broadcasted_iota(jnp.int32, (pblk, pblk // 128), 1)
    A = (iota_s == iota_i // 128).astype(f32)
    M = lax.dot_general(A, labL, (((1,), (0,)), ((), ())),
                        preferred_element_type=f32)
    lane_g = lax.broadcasted_iota(jnp.int32, (pblk, 128), 1)
    row_g = lax.broadcasted_iota(jnp.int32, (pblk, 128), 0)
    lab2 = jnp.sum(jnp.where(lane_g == row_g % 128, M, 0.0),
                   axis=1, keepdims=True)
    ones_c = jnp.ones((1, C), f32)
    sumexp = lax.dot_general(ones_c, jnp.exp(conf), dn, preferred_element_type=f32)
    iota_c = lax.broadcasted_iota(jnp.int32, (pblk, C), 1).astype(f32)
    onehot = iota_c == lab2
    gathered = lax.dot_general(ones_c, jnp.where(onehot, conf, 0.0), dn,
                               preferred_element_type=f32)
    zlab = lax.dot_general(jnp.ones((1, 1), f32), (lab2 == 0.0).astype(f32), dn,
                           preferred_element_type=f32)

    lane_i = lax.broadcasted_iota(jnp.int32, (1, pblk), 1)
    valid_l = (p * pblk + lane_i) < P
    pos_l = zlab < 0.5
    ce = jnp.log(sumexp) - gathered
    lc = jnp.where(valid_l & jnp.logical_not(pos_l), jnp.maximum(ce, 0.0), 0.0)
    lc_ref[...] = jnp.reshape(lc, (1, 1, 1, pblk))

    posmask = valid_l & pos_l
    np_blk = jnp.sum(jnp.where(posmask, 1.0, 0.0))
    nz_blk = jnp.sum(jnp.where(lc > 0.0, 1.0, 0.0))
    tot_blk = jnp.sum(lc)
    posce_blk = jnp.sum(jnp.where(posmask, ce, 0.0))

    nrow = pblk // 32
    posLf = (labL > 0.5).astype(f32)
    iota_j = lax.broadcasted_iota(jnp.int32, (128, 128), 0)
    iota_l4 = lax.broadcasted_iota(jnp.int32, (128, 128), 1) // 4
    parts = []
    for g in range(4):
        Rg = (iota_j == (32 * g + iota_l4)).astype(f32)
        parts.append(lax.dot_general(posLf, Rg, (((1,), (0,)), ((), ())),
                                     preferred_element_type=f32))
    posrep = jnp.reshape(jnp.stack(parts, axis=1), (nrow, 128))
    d = loc_ref[0] - gt_ref[0]
    ad = jnp.abs(d)
    sl1 = jnp.where(ad < 1.0, 0.5 * d * d, ad - 0.5)
    sl1_blk = jnp.sum(jnp.where(posrep > 0.5, sl1, 0.0))

    li = lax.broadcasted_iota(jnp.int32, (1, 128), 1)
    vals = jnp.reshape(
        jnp.where(li == 0, np_blk, 0.0)
        + jnp.where(li == 1, nz_blk, 0.0)
        + jnp.where(li == 2, tot_blk, 0.0)
        + jnp.where(li == 3, posce_blk, 0.0)
        + jnp.where(li == 4, sl1_blk, 0.0), (1, 1, 128))

    @pl.when(p == 0)
    def _init():
        st_ref[...] = vals

    @pl.when(p != 0)
    def _acc():
        st_ref[...] += vals


def _p2_body(lc_ref, st_ref, out_ref, *, B, P, Ppad):
    f32, i32 = jnp.float32, jnp.int32
    CH = Ppad if Ppad < 2048 else 2048
    NCH = Ppad // CH
    st = st_ref[...]
    npv = st[:, 0:1]
    Kf = jnp.minimum(float(_NEGPOS_RATIO) * npv, float(P - 1))
    Ki = Kf.astype(i32)

    def cnt_ge(mid):
        def body(c, acc):
            v = lc_ref[:, pl.ds(c * CH, CH)]
            ui = lax.bitcast_convert_type(v, i32)
            return acc + jnp.sum((ui >= mid).astype(i32), axis=1, keepdims=True)
        return lax.fori_loop(0, NCH, body, jnp.zeros((B, 1), i32))

    def bis(_, lohi):
        lo, hi = lohi
        mid = lo + (hi - lo) // 2
        ge = cnt_ge(mid) >= Ki
        return (jnp.where(ge, mid, lo), jnp.where(ge, hi, mid))
    lo, _ = lax.fori_loop(
        0, 31, bis,
        (jnp.zeros((B, 1), i32), jnp.full((B, 1), 2**31 - 1, i32)))
    t = lax.bitcast_convert_type(lo, f32)

    def body2(c, acc):
        s, cg = acc
        v = lc_ref[:, pl.ds(c * CH, CH)]
        gtm = v > t
        s = s + jnp.sum(jnp.where(gtm, v, 0.0), axis=1, keepdims=True)
        cg = cg + jnp.sum(gtm.astype(f32), axis=1, keepdims=True)
        return (s, cg)
    sgt, cgt = lax.fori_loop(0, NCH, body2,
                             (jnp.zeros((B, 1), f32), jnp.zeros((B, 1), f32)))
    S = jnp.where(Ki > 0, sgt + (Kf - cgt) * t, 0.0)

    N = jnp.sum(npv)
    loss = (jnp.sum(st[:, 3:4]) + jnp.sum(st[:, 4:5]) + jnp.sum(S)) / N
    out_ref[...] = jnp.reshape(loss, (1, 1))


def kernel(loc, conf, priors, loc_gt, labels):
    del priors
    B, P, C = conf.shape
    pblk = _PBLK
    NP = (P + pblk - 1) // pblk
    Ppad = NP * pblk

    labf = jnp.reshape(
        jnp.pad(labels[..., 0].astype(jnp.float32), ((0, 0), (0, Ppad - P))),
        (B, Ppad // 128, 128))
    locf = jnp.reshape(
        jnp.pad(jnp.reshape(loc, (B, P * 4)), ((0, 0), (0, (Ppad - P) * 4))),
        (B, Ppad // 32, 128))
    gtf = jnp.reshape(
        jnp.pad(jnp.reshape(loc_gt, (B, P * 4)), ((0, 0), (0, (Ppad - P) * 4))),
        (B, Ppad // 32, 128))

    lc, st = pl.pallas_call(
        functools.partial(_p1_body, P=P, C=C, pblk=pblk),
        grid=(B, NP),
        in_specs=[
            pl.BlockSpec((1, pblk, C), lambda b, p: (b, p, 0)),
            pl.BlockSpec((1, pblk // 128, 128), lambda b, p: (b, p, 0)),
            pl.BlockSpec((1, pblk // 32, 128), lambda b, p: (b, p, 0)),
            pl.BlockSpec((1, pblk // 32, 128), lambda b, p: (b, p, 0)),
        ],
        out_specs=[
            pl.BlockSpec((1, 1, 1, pblk), lambda b, p: (b, p, 0, 0)),
            pl.BlockSpec((1, 1, 128), lambda b, p: (b, 0, 0)),
        ],
        out_shape=[
            jax.ShapeDtypeStruct((B, NP, 1, pblk), jnp.float32),
            jax.ShapeDtypeStruct((B, 1, 128), jnp.float32),
        ],
        compiler_params=pltpu.CompilerParams(
            dimension_semantics=("arbitrary", "arbitrary")),
    )(conf, labf, locf, gtf)
    lc = jnp.reshape(lc, (B, Ppad))
    st = jnp.reshape(st, (B, 128))

    out = pl.pallas_call(
        functools.partial(_p2_body, B=B, P=P, Ppad=Ppad),
        out_shape=jax.ShapeDtypeStruct((1, 1), jnp.float32),
    )(lc, st)
    return jnp.reshape(out, ())
